# baseline (speedup 1.0000x reference)
import functools

import jax
import jax.numpy as jnp
from jax import lax
from jax.experimental import pallas as pl
from jax.experimental.pallas import tpu as pltpu
from jax.experimental.pallas import tpu_sc as plsc

NC = 2
NS = 16
LANES = 16
WORKERS = NC * NS
WIN = 256


def _vfull(val):
    return jnp.full((LANES,), val, dtype=jnp.int32)


def _tc_stage_body(depth, batch, scores_ref, out_ref, sem):
    copies = [
        pltpu.make_async_copy(
            scores_ref.at[:, pl.ds(((2 ** i - 1) // 128) * 128, WIN)],
            out_ref.at[pl.ds(i * batch, batch), :], sem)
        for i in range(depth)
    ]
    for c in copies:
        c.start()
    for c in copies:
        c.wait()


def _sc_gather_body(depth, lvl_pad, rows_pw, batch,
                    panel_hbm, ci_hbm, out_hbm, ci_v, idx_v, gat_v, sem):
    slots = rows_pw * lvl_pad
    r_bits = rows_pw.bit_length() - 1
    wid = lax.axis_index("s") * NC + lax.axis_index("c")
    wid_v = _vfull(wid)
    pltpu.sync_copy(ci_hbm, ci_v)
    vreg_id = lax.shift_right_logical(
        wid_v, _vfull((LANES // rows_pw).bit_length() - 1))
    c8 = _vfull(0)
    for k in range(batch // LANES):
        vk = ci_v[pl.ds(k * LANES, LANES)]
        d = lax.sub(vreg_id, _vfull(k))
        m = lax.sub(_vfull(1), lax.min(lax.mul(d, d), _vfull(1)))
        c8 = lax.add(c8, lax.mul(vk, m))
    iota = lax.iota(jnp.int32, LANES)
    r = jnp.bitwise_and(iota, _vfull(rows_pw - 1))
    lane0 = lax.mul(jnp.bitwise_and(wid_v, _vfull(LANES // rows_pw - 1)),
                    _vfull(rows_pw))
    c = lax.gather(
        c8, lax.add(lane0, r)[:, None],
        lax.GatherDimensionNumbers(offset_dims=(), collapsed_slice_dims=(0,),
                                   start_index_map=(0,)),
        slice_sizes=(1,), mode=lax.GatherScatterMode.PROMISE_IN_BOUNDS)
    grow = lax.add(lax.mul(wid_v, _vfull(rows_pw)), r)
    for j in range(slots // LANES):
        s = lax.add(iota, _vfull(j * LANES))
        lvl = lax.shift_right_logical(s, _vfull(r_bits))
        shift = lax.max(lax.sub(_vfull(depth - 1), lvl), _vfull(0))
        bit = jnp.bitwise_and(lax.shift_right_logical(c, shift), _vfull(1))
        t = jnp.bitwise_and(lax.sub(lax.shift_left(_vfull(1), lvl), _vfull(1)),
                            _vfull(127))
        prow = lax.add(lax.mul(lvl, _vfull(batch)), grow)
        flat = lax.add(lax.mul(prow, _vfull(WIN)), lax.add(t, bit))
        flat = lax.select(lax.lt(lvl, _vfull(depth)), flat, _vfull(0))
        idx_v[pl.ds(j * LANES, LANES)] = flat
    pltpu.async_copy(panel_hbm.at[idx_v], gat_v, sem).wait()
    pltpu.sync_copy(gat_v, out_hbm.at[pl.ds(wid * slots, slots)])


def _tc_reduce_body(depth, lvl_pad, rows_pw, batch, g_ref, o_ref):
    x = g_ref[...]
    slot = lax.broadcasted_iota(jnp.int32, x.shape, 1)
    lvl = slot // rows_pw
    sp = jnp.where(lvl < depth, jax.nn.softplus(-x), 0.0)
    o_ref[...] = (jnp.sum(sp) / batch).reshape(1, 1)


def kernel(scores, class_indices):
    batch, vocab = scores.shape
    depth = max(1, (vocab - 1).bit_length())
    rows_pw = batch // WORKERS
    lvl_pad = depth
    while (rows_pw * lvl_pad) % LANES:
        lvl_pad += 1
    slots = rows_pw * lvl_pad

    panel = pl.pallas_call(
        functools.partial(_tc_stage_body, depth, batch),
        in_specs=[pl.BlockSpec(memory_space=pltpu.MemorySpace.HBM)],
        out_specs=pl.BlockSpec(memory_space=pltpu.MemorySpace.HBM),
        out_shape=jax.ShapeDtypeStruct((depth * batch, WIN), jnp.float32),
        scratch_shapes=[pltpu.SemaphoreType.DMA],
    )(scores)

    mesh = plsc.VectorSubcoreMesh(core_axis_name="c", subcore_axis_name="s",
                                  num_cores=NC, num_subcores=NS)
    sc_gather = pl.kernel(
        functools.partial(_sc_gather_body, depth, lvl_pad, rows_pw, batch),
        out_type=jax.ShapeDtypeStruct((WORKERS * slots,), jnp.float32),
        mesh=mesh,
        scratch_types=[
            pltpu.VMEM((batch,), jnp.int32),
            pltpu.VMEM((slots,), jnp.int32),
            pltpu.VMEM((slots,), jnp.float32),
            pltpu.SemaphoreType.DMA,
        ],
    )
    gathered = sc_gather(panel.reshape(-1), class_indices)

    loss = pl.pallas_call(
        functools.partial(_tc_reduce_body, depth, lvl_pad, rows_pw, batch),
        out_shape=jax.ShapeDtypeStruct((1, 1), jnp.float32),
    )(gathered.reshape(WORKERS, slots))
    return loss[0, 0]


# speedup vs baseline: 5.6656x; 5.6656x over previous
import functools

import jax
import jax.numpy as jnp
from jax import lax
from jax.experimental import pallas as pl
from jax.experimental.pallas import tpu as pltpu
from jax.experimental.pallas import tpu_sc as plsc

NC = 2
NS = 16
LANES = 16
WORKERS = NC * NS


def _vfull(val):
    return jnp.full((LANES,), val, dtype=jnp.int32)


def _sc_gather_body(depth, lvl_pad, rows_pw, batch,
                    flat_hbm, ci_hbm, out_hbm, ci_v, idx_v, gat_v, sem):
    slots = rows_pw * lvl_pad
    r_bits = rows_pw.bit_length() - 1
    wid = lax.axis_index("s") * NC + lax.axis_index("c")
    wid_v = _vfull(wid)
    pltpu.sync_copy(ci_hbm, ci_v)
    vreg_id = lax.shift_right_logical(
        wid_v, _vfull((LANES // rows_pw).bit_length() - 1))
    c8 = _vfull(0)
    for k in range(batch // LANES):
        vk = ci_v[pl.ds(k * LANES, LANES)]
        d = lax.sub(vreg_id, _vfull(k))
        m = lax.sub(_vfull(1), lax.min(lax.mul(d, d), _vfull(1)))
        c8 = lax.add(c8, lax.mul(vk, m))
    iota = lax.iota(jnp.int32, LANES)
    r = jnp.bitwise_and(iota, _vfull(rows_pw - 1))
    lane0 = lax.mul(jnp.bitwise_and(wid_v, _vfull(LANES // rows_pw - 1)),
                    _vfull(rows_pw))
    c = lax.gather(
        c8, lax.add(lane0, r)[:, None],
        lax.GatherDimensionNumbers(offset_dims=(), collapsed_slice_dims=(0,),
                                   start_index_map=(0,)),
        slice_sizes=(1,), mode=lax.GatherScatterMode.PROMISE_IN_BOUNDS)
    grow = lax.add(lax.mul(wid_v, _vfull(rows_pw)), r)
    for j in range(slots // LANES):
        s = lax.add(iota, _vfull(j * LANES))
        lvl = lax.shift_right_logical(s, _vfull(r_bits))
        shift = lax.max(lax.sub(_vfull(depth - 1), lvl), _vfull(0))
        bit = jnp.bitwise_and(lax.shift_right_logical(c, shift), _vfull(1))
        col = lax.add(lax.sub(lax.shift_left(_vfull(1), lvl), _vfull(1)), bit)
        flat = lax.add(lax.mul(col, _vfull(batch)), grow)
        flat = lax.select(lax.lt(lvl, _vfull(depth)), flat, _vfull(0))
        idx_v[pl.ds(j * LANES, LANES)] = flat
    pltpu.async_copy(flat_hbm.at[idx_v], gat_v, sem).wait()
    pltpu.sync_copy(gat_v, out_hbm.at[pl.ds(wid * slots, slots)])


def _tc_reduce_body(depth, lvl_pad, rows_pw, batch, g_ref, o_ref):
    x = g_ref[...]
    slot = lax.broadcasted_iota(jnp.int32, x.shape, 1)
    lvl = slot // rows_pw
    sp = jnp.where(lvl < depth, jax.nn.softplus(-x), 0.0)
    o_ref[...] = (jnp.sum(sp) / batch).reshape(1, 1)


def kernel(scores, class_indices):
    batch, vocab = scores.shape
    depth = max(1, (vocab - 1).bit_length())
    rows_pw = batch // WORKERS
    lvl_pad = depth
    while (rows_pw * lvl_pad) % LANES:
        lvl_pad += 1
    slots = rows_pw * lvl_pad

    mesh = plsc.VectorSubcoreMesh(core_axis_name="c", subcore_axis_name="s",
                                  num_cores=NC, num_subcores=NS)
    sc_gather = pl.kernel(
        functools.partial(_sc_gather_body, depth, lvl_pad, rows_pw, batch),
        out_type=jax.ShapeDtypeStruct((WORKERS * slots,), jnp.float32),
        mesh=mesh,
        scratch_types=[
            pltpu.VMEM((batch,), jnp.int32),
            pltpu.VMEM((slots,), jnp.int32),
            pltpu.VMEM((slots,), jnp.float32),
            pltpu.SemaphoreType.DMA,
        ],
    )
    gathered = sc_gather(scores.T.reshape(-1), class_indices)

    loss = pl.pallas_call(
        functools.partial(_tc_reduce_body, depth, lvl_pad, rows_pw, batch),
        out_shape=jax.ShapeDtypeStruct((1, 1), jnp.float32),
    )(gathered.reshape(WORKERS, slots))
    return loss[0, 0]


# speedup vs baseline: 6.0920x; 1.0752x over previous
import functools

import jax
import jax.numpy as jnp
from jax import lax
from jax.experimental import pallas as pl
from jax.experimental.pallas import tpu as pltpu
from jax.experimental.pallas import tpu_sc as plsc

NC = 2
NS = 16
LANES = 16
WORKERS = NC * NS


def _vfull(val):
    return jnp.full((LANES,), val, dtype=jnp.int32)


def _sc_gather_body(depth, lvl_pad, rows_pw, batch,
                    flat_hbm, ci_hbm, out_hbm, ci_v, idx_v, gat_v, sem):
    slots = rows_pw * lvl_pad
    r_bits = rows_pw.bit_length() - 1
    wid = lax.axis_index("s") * NC + lax.axis_index("c")
    wid_v = _vfull(wid)
    vpw = LANES // rows_pw
    off = pl.multiple_of((wid // vpw) * LANES, LANES)
    pltpu.sync_copy(ci_hbm.at[pl.ds(off, LANES)], ci_v)
    c16 = ci_v[pl.ds(0, LANES)]
    iota = lax.iota(jnp.int32, LANES)
    r = jnp.bitwise_and(iota, _vfull(rows_pw - 1))
    lane0 = lax.mul(jnp.bitwise_and(wid_v, _vfull(vpw - 1)),
                    _vfull(rows_pw))
    c = lax.gather(
        c16, lax.add(lane0, r)[:, None],
        lax.GatherDimensionNumbers(offset_dims=(), collapsed_slice_dims=(0,),
                                   start_index_map=(0,)),
        slice_sizes=(1,), mode=lax.GatherScatterMode.PROMISE_IN_BOUNDS)
    grow = lax.add(lax.mul(wid_v, _vfull(rows_pw)), r)
    for j in range(slots // LANES):
        s = lax.add(iota, _vfull(j * LANES))
        lvl = lax.shift_right_logical(s, _vfull(r_bits))
        shift = lax.max(lax.sub(_vfull(depth - 1), lvl), _vfull(0))
        bit = jnp.bitwise_and(lax.shift_right_logical(c, shift), _vfull(1))
        col = lax.add(lax.sub(lax.shift_left(_vfull(1), lvl), _vfull(1)), bit)
        flat = lax.add(lax.mul(col, _vfull(batch)), grow)
        flat = lax.select(lax.lt(lvl, _vfull(depth)), flat, _vfull(0))
        idx_v[pl.ds(j * LANES, LANES)] = flat
    pltpu.async_copy(flat_hbm.at[idx_v], gat_v, sem).wait()
    pltpu.sync_copy(gat_v, out_hbm.at[pl.ds(wid * slots, slots)])


def _tc_reduce_body(depth, lvl_pad, rows_pw, batch, g_ref, o_ref):
    x = g_ref[...]
    slot = lax.iota(jnp.int32, x.shape[0]) % (rows_pw * lvl_pad)
    lvl = slot // rows_pw
    sp = jnp.where(lvl < depth, jax.nn.softplus(-x), 0.0)
    o_ref[...] = (jnp.sum(sp) / batch).reshape(1, 1)


def kernel(scores, class_indices):
    batch, vocab = scores.shape
    depth = max(1, (vocab - 1).bit_length())
    rows_pw = batch // WORKERS
    lvl_pad = depth
    while (rows_pw * lvl_pad) % LANES:
        lvl_pad += 1
    slots = rows_pw * lvl_pad

    mesh = plsc.VectorSubcoreMesh(core_axis_name="c", subcore_axis_name="s",
                                  num_cores=NC, num_subcores=NS)
    sc_gather = pl.kernel(
        functools.partial(_sc_gather_body, depth, lvl_pad, rows_pw, batch),
        out_type=jax.ShapeDtypeStruct((WORKERS * slots,), jnp.float32),
        mesh=mesh,
        scratch_types=[
            pltpu.VMEM((LANES,), jnp.int32),
            pltpu.VMEM((slots,), jnp.int32),
            pltpu.VMEM((slots,), jnp.float32),
            pltpu.SemaphoreType.DMA,
        ],
    )
    gathered = sc_gather(scores.T.reshape(-1), class_indices)

    loss = pl.pallas_call(
        functools.partial(_tc_reduce_body, depth, lvl_pad, rows_pw, batch),
        out_shape=jax.ShapeDtypeStruct((1, 1), jnp.float32),
    )(gathered)
    return loss[0, 0]
